# baseline (speedup 1.0000x reference)
import functools

import jax
import jax.numpy as jnp
from jax import lax
from jax.experimental import pallas as pl
from jax.experimental.pallas import tpu as pltpu
from jax.experimental.pallas import tpu_sc as plsc

_NROWS = 2048
_ROW = 32768
_WIN = 33792
_RPW = 64


def _sc_body(flat_hbm, out_hbm, win, sem):
    c = lax.axis_index("c")
    s = lax.axis_index("s")
    wid = s * 2 + c

    lo = pl.multiple_of(31744 - 1024 * wid, 1024)
    pltpu.sync_copy(flat_hbm.at[pl.ds(lo, _WIN)], win)

    base = _RPW * wid
    for chunk in range(0, _RPW, 16):
        copies = [
            pltpu.make_async_copy(
                win.at[pl.ds(1008 - 16 * r, _ROW)],
                out_hbm.at[pl.ds(pl.multiple_of((base + r) * _ROW, _ROW), _ROW)],
                sem,
            )
            for r in range(chunk, chunk + 16)
        ]
        for cp in copies:
            cp.start()
        for cp in copies:
            cp.wait()


@functools.partial(jax.jit, static_argnames=())
def _sc_write(flat1d):
    mesh = plsc.VectorSubcoreMesh(core_axis_name="c", subcore_axis_name="s")
    run = functools.partial(
        pl.kernel,
        mesh=mesh,
        out_type=jax.ShapeDtypeStruct((_NROWS * _ROW,), jnp.float32),
        scratch_types=[
            pltpu.VMEM((_WIN,), jnp.float32),
            pltpu.SemaphoreType.DMA,
        ],
    )(_sc_body)
    return run(flat1d)


def kernel(inputs, rel_embeddings):
    del inputs
    flat = jnp.flip(rel_embeddings, axis=0).reshape(-1)
    flat = jnp.concatenate([flat, jnp.zeros((16,), flat.dtype)])
    out = _sc_write(flat)
    return out.reshape(2048, 2048, 16)


# speedup vs baseline: 2.7504x; 2.7504x over previous
import functools

import jax
import jax.numpy as jnp
from jax import lax
from jax.experimental import pallas as pl
from jax.experimental.pallas import tpu as pltpu
from jax.experimental.pallas import tpu_sc as plsc

_RPW = 64


def _sc_body(q_hbm, out_hbm, q_sh, sem):
    c = lax.axis_index("c")
    s = lax.axis_index("s")
    wid = s * 2 + c

    @pl.when(s == 0)
    def _():
        pltpu.sync_copy(q_hbm, q_sh)

    plsc.subcore_barrier()

    base = _RPW * wid
    for chunk in range(0, _RPW, 16):
        copies = [
            pltpu.make_async_copy(
                q_sh.at[7 - (r % 8), pl.ds(255 - 8 * wid - (r // 8), 256), :],
                out_hbm.at[base + r],
                sem,
            )
            for r in range(chunk, chunk + 16)
        ]
        for cp in copies:
            cp.start()
        for cp in copies:
            cp.wait()


def _sc_write(q):
    mesh = plsc.VectorSubcoreMesh(core_axis_name="c", subcore_axis_name="s")
    run = functools.partial(
        pl.kernel,
        mesh=mesh,
        out_type=jax.ShapeDtypeStruct((2048, 256, 128), jnp.float32),
        scratch_types=[
            pltpu.VMEM_SHARED((8, 512, 128), jnp.float32),
            pltpu.SemaphoreType.DMA,
        ],
    )(_sc_body)
    return run(q)


def kernel(inputs, rel_embeddings):
    del inputs
    flat = jnp.flip(rel_embeddings, axis=0).reshape(-1)
    flat = jnp.concatenate([flat, jnp.zeros((128,), flat.dtype)])
    q = jnp.stack(
        [
            jax.lax.dynamic_slice(flat, (16 * p,), (65536,)).reshape(512, 128)
            for p in range(8)
        ]
    )
    out = _sc_write(q)
    return out.reshape(2048, 2048, 16)


# speedup vs baseline: 2.7579x; 1.0027x over previous
import functools

import jax
import jax.numpy as jnp
from jax import lax
from jax.experimental import pallas as pl
from jax.experimental.pallas import tpu as pltpu
from jax.experimental.pallas import tpu_sc as plsc

_RPW = 64


def _build_body(f_ref, q_ref):
    f = f_ref[...]
    for p in range(8):
        if p == 0:
            q_ref[0] = f[0:512, :]
        else:
            q_ref[p] = jnp.concatenate(
                [f[0:512, 16 * p:], f[1:513, : 16 * p]], axis=1
            )


def _build_phase_table(f2d):
    return pl.pallas_call(
        _build_body,
        in_specs=[pl.BlockSpec(memory_space=pltpu.MemorySpace.VMEM)],
        out_specs=pl.BlockSpec(memory_space=pltpu.MemorySpace.VMEM),
        out_shape=jax.ShapeDtypeStruct((8, 512, 128), jnp.float32),
    )(f2d)


def _sc_body(q_hbm, out_hbm, q_sh, sem):
    c = lax.axis_index("c")
    s = lax.axis_index("s")
    wid = s * 2 + c

    @pl.when(s == 0)
    def _():
        pltpu.sync_copy(q_hbm, q_sh)

    plsc.subcore_barrier()

    base = _RPW * wid
    for chunk in range(0, _RPW, 16):
        copies = [
            pltpu.make_async_copy(
                q_sh.at[7 - (r % 8), pl.ds(255 - 8 * wid - (r // 8), 256), :],
                out_hbm.at[base + r],
                sem,
            )
            for r in range(chunk, chunk + 16)
        ]
        for cp in copies:
            cp.start()
        for cp in copies:
            cp.wait()


def _sc_write(q):
    mesh = plsc.VectorSubcoreMesh(core_axis_name="c", subcore_axis_name="s")
    run = functools.partial(
        pl.kernel,
        mesh=mesh,
        out_type=jax.ShapeDtypeStruct((2048, 256, 128), jnp.float32),
        scratch_types=[
            pltpu.VMEM_SHARED((8, 512, 128), jnp.float32),
            pltpu.SemaphoreType.DMA,
        ],
    )(_sc_body)
    return run(q)


def kernel(inputs, rel_embeddings):
    del inputs
    flat = jnp.flip(rel_embeddings, axis=0).reshape(-1)
    f2d = jnp.concatenate([flat, jnp.zeros((144,), flat.dtype)]).reshape(513, 128)
    q = _build_phase_table(f2d)
    out = _sc_write(q)
    return out.reshape(2048, 2048, 16)
